# baseline (speedup 1.0000x reference)
import functools

import jax
import jax.numpy as jnp
import numpy as np
from jax.experimental import pallas as pl
from jax.experimental.pallas import tpu as pltpu

_H0, _W0 = 48, 48
_HT, _WT = 96, 96
_BB = 4


def _wresize_mat() -> np.ndarray:
    C = np.zeros((_W0, _WT), dtype=np.float64)
    for o in range(_WT):
        s = 0.5 * o - 0.25
        sc = min(max(s, 0.0), _W0 - 1.0)
        i0 = min(int(np.floor(sc)), _W0 - 2)
        w1 = sc - i0
        C[i0, o] += 1.0 - w1
        C[i0 + 1, o] += w1
    return C.astype(np.float32)


def _src(p):
    return jnp.clip(0.5 * p.astype(jnp.float32) - 0.25, 0.0, _H0 - 1.0)


def _w2d(py, px, shape):
    nlead = len(shape) - 2
    exp = (Ellipsis,) + (None,) * 2
    sy = _src(py)[exp]
    sx = _src(px)[exp]
    f32 = jnp.float32
    ri = jax.lax.broadcasted_iota(jnp.int32, shape, nlead).astype(f32)
    ci = jax.lax.broadcasted_iota(jnp.int32, shape, nlead + 1).astype(f32)
    wy = jnp.maximum(1.0 - jnp.abs(ri - sy), 0.0)
    wx = jnp.maximum(1.0 - jnp.abs(ci - sx), 0.0)
    return wy * wx


def _hresize_parity(a):
    up = jnp.concatenate([a[..., :1, :], a[..., :-1, :]], axis=-2)
    dn = jnp.concatenate([a[..., 1:, :], a[..., -1:, :]], axis=-2)
    even = 0.75 * a + 0.25 * up
    odd = 0.75 * a + 0.25 * dn
    return jnp.stack([even, odd], axis=-3)


def _decode_kernel(hm_ref, ct_ref, rg_ref, of_ref, c_ref, out_ref,
                   *, nj, bb):
    f32 = jnp.float32
    C = c_ref[...]
    nc = nj + 1

    hm = hm_ref[...]
    ct = ct_ref[...]
    rg = rg_ref[...]
    of = of_ref[...]

    x_in = jnp.concatenate([ct, hm], axis=1)
    x2 = x_in.reshape(bb * nc * _H0, _W0)
    cb = C.astype(jnp.bfloat16)
    x_1 = x2.astype(jnp.bfloat16)
    r_1 = x2 - x_1.astype(f32)
    x_2 = r_1.astype(jnp.bfloat16)
    x_3 = (r_1 - x_2.astype(f32)).astype(jnp.bfloat16)
    mm = lambda u: jnp.dot(u, cb, preferred_element_type=f32)
    a = mm(x_1) + (mm(x_2) + mm(x_3))
    f = _hresize_parity(a.reshape(bb, nc, _H0, _WT))
    ct_r = f[:, 0]
    hm_r = f[:, 1:]

    pshape = (2, _H0, _WT)
    yrow = (jax.lax.broadcasted_iota(jnp.int32, pshape, 1) * 2
            + jax.lax.broadcasted_iota(jnp.int32, pshape, 0))
    li = yrow * _WT + jax.lax.broadcasted_iota(jnp.int32, pshape, 2)
    big = _HT * _WT

    m = jnp.max(ct_r, axis=(1, 2, 3))
    idx = jnp.min(jnp.where(ct_r == m[:, None, None, None], li[None], big),
                  axis=(1, 2, 3))
    cy = idx // _WT
    cx = idx - cy * _WT

    wc = _w2d(cy, cx, (bb, _H0, _W0))
    rxy = jnp.sum(rg * wc[:, None], axis=(2, 3))
    rxy = rxy.reshape(bb, nj, 2)
    reg_x = jnp.clip(cx.astype(f32)[:, None] + rxy[:, :, 0] + 0.5,
                     0.0, _WT - 1.0)
    reg_y = jnp.clip(cy.astype(f32)[:, None] + rxy[:, :, 1] + 0.5,
                     0.0, _HT - 1.0)

    yrf = yrow.astype(f32)
    xrf = jax.lax.broadcasted_iota(jnp.int32, pshape, 2).astype(f32)
    d2 = ((yrf[None, None] - reg_y[:, :, None, None, None]) ** 2 + 1e-9
          + (xrf[None, None] - reg_x[:, :, None, None, None]) ** 2)
    t = hm_r * jax.lax.rsqrt(d2)
    m2 = jnp.max(t, axis=(2, 3, 4))
    idx2 = jnp.min(jnp.where(t == m2[:, :, None, None, None],
                             li[None, None], big), axis=(2, 3, 4))
    jy = idx2 // _WT
    jx = idx2 - jy * _WT
    jyf = jy.astype(f32)
    jxf = jx.astype(f32)
    d2p = (jyf - reg_y) ** 2 + 1e-9 + (jxf - reg_x) ** 2
    score = m2 * jnp.sqrt(d2p)

    wj = _w2d(jy, jx, (bb, nj, _H0, _W0))
    oxy = jnp.sum(of.reshape(bb, nj, 2, _H0, _W0) * wj[:, :, None],
                  axis=(3, 4))

    xv = (jxf + oxy[:, :, 0]) / float(_WT)
    yv = (jyf + oxy[:, :, 1]) / float(_HT)
    lane = jax.lax.broadcasted_iota(jnp.int32, (bb, 3 * nj), 1)
    jmap = lane // 3
    cmap = lane - 3 * jmap
    xg = jnp.take_along_axis(xv, jmap, axis=1)
    yg = jnp.take_along_axis(yv, jmap, axis=1)
    sg = jnp.take_along_axis(score, jmap, axis=1)
    out = jnp.where(cmap == 0, xg, jnp.where(cmap == 1, yg, sg))
    out_ref[:, 0, :] = out


def kernel(hm, ct, rg, of):
    B, nj = hm.shape[0], hm.shape[1]
    bb = _BB
    Cm = jnp.asarray(_wresize_mat())
    spec3 = lambda c: pl.BlockSpec((bb, c, _H0, _W0), lambda b: (b, 0, 0, 0))
    out = pl.pallas_call(
        functools.partial(_decode_kernel, nj=nj, bb=bb),
        grid=(B // bb,),
        in_specs=[
            spec3(nj),
            spec3(1),
            spec3(2 * nj),
            spec3(2 * nj),
            pl.BlockSpec((_W0, _WT), lambda b: (0, 0)),
        ],
        out_specs=pl.BlockSpec((bb, 1, 3 * nj), lambda b: (b, 0, 0)),
        out_shape=jax.ShapeDtypeStruct((B, 1, 3 * nj), jnp.float32),
        compiler_params=pltpu.CompilerParams(
            dimension_semantics=("arbitrary",),
        ),
    )(hm, ct, rg, of, Cm)
    return out.reshape(B, 3 * nj)


# speedup vs baseline: 1.2232x; 1.2232x over previous
import functools

import jax
import jax.numpy as jnp
import numpy as np
from jax.experimental import pallas as pl
from jax.experimental.pallas import tpu as pltpu

_H0, _W0 = 48, 48
_HT, _WT = 96, 96
_BB = 4


def _wresize_mat() -> np.ndarray:
    C = np.zeros((_W0, _WT), dtype=np.float64)
    for o in range(_WT):
        s = 0.5 * o - 0.25
        sc = min(max(s, 0.0), _W0 - 1.0)
        i0 = min(int(np.floor(sc)), _W0 - 2)
        w1 = sc - i0
        C[i0, o] += 1.0 - w1
        C[i0 + 1, o] += w1
    return C.astype(np.float32)


def _src(p):
    return jnp.clip(0.5 * p.astype(jnp.float32) - 0.25, 0.0, _H0 - 1.0)


def _w2d(py, px, shape):
    nlead = len(shape) - 2
    exp = (Ellipsis,) + (None,) * 2
    sy = _src(py)[exp]
    sx = _src(px)[exp]
    f32 = jnp.float32
    ri = jax.lax.broadcasted_iota(jnp.int32, shape, nlead).astype(f32)
    ci = jax.lax.broadcasted_iota(jnp.int32, shape, nlead + 1).astype(f32)
    wy = jnp.maximum(1.0 - jnp.abs(ri - sy), 0.0)
    wx = jnp.maximum(1.0 - jnp.abs(ci - sx), 0.0)
    return wy * wx


def _hresize_parity(a):
    up = jnp.concatenate([a[..., :1, :], a[..., :-1, :]], axis=-2)
    dn = jnp.concatenate([a[..., 1:, :], a[..., -1:, :]], axis=-2)
    even = 0.75 * a + 0.25 * up
    odd = 0.75 * a + 0.25 * dn
    return jnp.stack([even, odd], axis=-3)


def _decode_kernel(hm_ref, ct_ref, rg_ref, of_ref, c_ref, out_ref,
                   *, nj, bb):
    f32 = jnp.float32
    C = c_ref[...]
    nc = nj + 1

    hm = hm_ref[...].reshape(bb, nj, _H0, _W0)
    ct = ct_ref[...].reshape(bb, 1, _H0, _W0)
    rg = rg_ref[...].reshape(bb, 2 * nj, _H0, _W0)
    of = of_ref[...].reshape(bb, 2 * nj, _H0, _W0)

    x_in = jnp.concatenate([ct, hm], axis=1)
    x2 = x_in.reshape(bb * nc * _H0, _W0)
    cb = C.astype(jnp.bfloat16)
    x_1 = x2.astype(jnp.bfloat16)
    r_1 = x2 - x_1.astype(f32)
    x_2 = r_1.astype(jnp.bfloat16)
    x_3 = (r_1 - x_2.astype(f32)).astype(jnp.bfloat16)
    mm = lambda u: jnp.dot(u, cb, preferred_element_type=f32)
    a = mm(x_1) + (mm(x_2) + mm(x_3))
    f = _hresize_parity(a.reshape(bb, nc, _H0, _WT))
    ct_r = f[:, 0]
    hm_r = f[:, 1:]

    pshape = (2, _H0, _WT)
    yrow = (jax.lax.broadcasted_iota(jnp.int32, pshape, 1) * 2
            + jax.lax.broadcasted_iota(jnp.int32, pshape, 0))
    li = yrow * _WT + jax.lax.broadcasted_iota(jnp.int32, pshape, 2)
    big = _HT * _WT

    m = jnp.max(ct_r, axis=(1, 2, 3))
    idx = jnp.min(jnp.where(ct_r == m[:, None, None, None], li[None], big),
                  axis=(1, 2, 3))
    cy = idx // _WT
    cx = idx - cy * _WT

    wc = _w2d(cy, cx, (bb, _H0, _W0))
    rxy = jnp.sum(rg * wc[:, None], axis=(2, 3))
    rxy = rxy.reshape(bb, nj, 2)
    reg_x = jnp.clip(cx.astype(f32)[:, None] + rxy[:, :, 0] + 0.5,
                     0.0, _WT - 1.0)
    reg_y = jnp.clip(cy.astype(f32)[:, None] + rxy[:, :, 1] + 0.5,
                     0.0, _HT - 1.0)

    yrf = yrow.astype(f32)
    xrf = jax.lax.broadcasted_iota(jnp.int32, pshape, 2).astype(f32)
    d2 = ((yrf[None, None] - reg_y[:, :, None, None, None]) ** 2 + 1e-9
          + (xrf[None, None] - reg_x[:, :, None, None, None]) ** 2)
    t = hm_r * jax.lax.rsqrt(d2)
    m2 = jnp.max(t, axis=(2, 3, 4))
    idx2 = jnp.min(jnp.where(t == m2[:, :, None, None, None],
                             li[None, None], big), axis=(2, 3, 4))
    jy = idx2 // _WT
    jx = idx2 - jy * _WT
    jyf = jy.astype(f32)
    jxf = jx.astype(f32)
    d2p = (jyf - reg_y) ** 2 + 1e-9 + (jxf - reg_x) ** 2
    score = m2 * jnp.sqrt(d2p)

    wj = _w2d(jy, jx, (bb, nj, _H0, _W0))
    oxy = jnp.sum(of.reshape(bb, nj, 2, _H0, _W0) * wj[:, :, None],
                  axis=(3, 4))

    xv = (jxf + oxy[:, :, 0]) / float(_WT)
    yv = (jyf + oxy[:, :, 1]) / float(_HT)
    lane = jax.lax.broadcasted_iota(jnp.int32, (bb, 3 * nj), 1)
    jmap = lane // 3
    cmap = lane - 3 * jmap
    xg = jnp.take_along_axis(xv, jmap, axis=1)
    yg = jnp.take_along_axis(yv, jmap, axis=1)
    sg = jnp.take_along_axis(score, jmap, axis=1)
    out = jnp.where(cmap == 0, xg, jnp.where(cmap == 1, yg, sg))
    out_ref[:, 0, :] = out


def kernel(hm, ct, rg, of):
    B, nj = hm.shape[0], hm.shape[1]
    bb = _BB
    Cm = jnp.asarray(_wresize_mat())
    spec2 = lambda c: pl.BlockSpec((bb * c * _H0, _W0), lambda b: (b, 0))
    out = pl.pallas_call(
        functools.partial(_decode_kernel, nj=nj, bb=bb),
        grid=(B // bb,),
        in_specs=[
            spec2(nj),
            spec2(1),
            spec2(2 * nj),
            spec2(2 * nj),
            pl.BlockSpec((_W0, _WT), lambda b: (0, 0)),
        ],
        out_specs=pl.BlockSpec((bb, 1, 3 * nj), lambda b: (b, 0, 0)),
        out_shape=jax.ShapeDtypeStruct((B, 1, 3 * nj), jnp.float32),
        compiler_params=pltpu.CompilerParams(
            dimension_semantics=("arbitrary",),
        ),
    )(hm.reshape(B * nj * _H0, _W0), ct.reshape(B * _H0, _W0),
      rg.reshape(B * 2 * nj * _H0, _W0), of.reshape(B * 2 * nj * _H0, _W0),
      Cm)
    return out.reshape(B, 3 * nj)
